# initial kernel scaffold (unmeasured)
import jax
import jax.numpy as jnp
from jax import lax
from jax.experimental import pallas as pl
from jax.experimental.pallas import tpu as pltpu

N_DEV = 32
TC = 64


def kernel(x, A, B, C):
    Bb, S, D = x.shape
    N = B.shape[-1]

    def body(x_ref, A_ref, B_ref, C_ref, y_ref, comm_ref, send_sem, recv_sem):
        my = lax.axis_index("i")
        right = lax.rem(my + 1, N_DEV)

        dA = jnp.exp(A_ref[:, :]).T

        def step(t, h):
            x_t = x_ref[:, pl.ds(t, 1), :][:, 0, :]
            B_t = B_ref[:, pl.ds(t, 1), :][:, 0, :]
            C_t = C_ref[:, pl.ds(t, 1), :][:, 0, :]
            h = h * dA[None] + x_t[:, None, :] * B_t[:, :, None]
            y_t = jnp.sum(h * C_t[:, :, None], axis=1)
            y_ref[:, pl.ds(t, 1), :] = y_t[:, None, :]
            return h

        h0 = jnp.zeros((Bb, N, D), jnp.float32)
        h_last = lax.fori_loop(0, S, step, h0)

        comm_ref[0] = h_last
        rdma = pltpu.make_async_remote_copy(
            src_ref=comm_ref.at[0],
            dst_ref=comm_ref.at[1],
            send_sem=send_sem,
            recv_sem=recv_sem,
            device_id=(right,),
            device_id_type=pl.DeviceIdType.MESH,
        )
        rdma.start()
        rdma.wait()

        h_in = jnp.where(my == 0, 0.0, comm_ref[1])

        def cstep(t, g):
            g = g * dA[None]
            C_t = C_ref[:, pl.ds(t, 1), :][:, 0, :]
            y_t = y_ref[:, pl.ds(t, 1), :][:, 0, :]
            y_ref[:, pl.ds(t, 1), :] = (
                y_t + jnp.sum(g * C_t[:, :, None], axis=1)
            )[:, None, :]
            return g

        lax.fori_loop(0, TC, cstep, h_in)

    return pl.pallas_call(
        body,
        out_shape=jax.ShapeDtypeStruct((Bb, S, D), jnp.float32),
        in_specs=[
            pl.BlockSpec(memory_space=pltpu.VMEM),
            pl.BlockSpec(memory_space=pltpu.VMEM),
            pl.BlockSpec(memory_space=pltpu.VMEM),
            pl.BlockSpec(memory_space=pltpu.VMEM),
        ],
        out_specs=pl.BlockSpec(memory_space=pltpu.VMEM),
        scratch_shapes=[
            pltpu.VMEM((2, Bb, N, D), jnp.float32),
            pltpu.SemaphoreType.DMA,
            pltpu.SemaphoreType.DMA,
        ],
        compiler_params=pltpu.CompilerParams(collective_id=0),
    )(x, A, B, C)


# baseline (device time: 116037 ns/iter reference)
import jax
import jax.numpy as jnp
from jax import lax
from jax.experimental import pallas as pl
from jax.experimental.pallas import tpu as pltpu

N_DEV = 32
TC = 64


def kernel(x, A, B, C):
    Bb, S, D = x.shape
    N = B.shape[-1]

    def body(x_ref, A_ref, B_ref, C_ref, y_ref, comm_ref, send_sem, recv_sem):
        my = lax.axis_index("i")
        right = lax.rem(my + 1, N_DEV)

        dA = jnp.exp(A_ref[:, :]).T

        def step(t, h):
            x_t = x_ref[:, pl.ds(t, 1), :][:, 0, :]
            B_t = B_ref[:, pl.ds(t, 1), :][:, 0, :]
            C_t = C_ref[:, pl.ds(t, 1), :][:, 0, :]
            h = h * dA[None] + x_t[:, None, :] * B_t[:, :, None]
            y_t = jnp.sum(h * C_t[:, :, None], axis=1)
            y_ref[:, pl.ds(t, 1), :] = y_t[:, None, :]
            return h

        h0 = jnp.zeros((Bb, N, D), jnp.float32)
        h_last = lax.fori_loop(0, S, step, h0)

        comm_ref[0] = h_last
        rdma = pltpu.make_async_remote_copy(
            src_ref=comm_ref.at[0],
            dst_ref=comm_ref.at[1],
            send_sem=send_sem,
            recv_sem=recv_sem,
            device_id=(right,),
            device_id_type=pl.DeviceIdType.MESH,
        )
        rdma.start()
        rdma.wait()

        h_in = jnp.where(my == 0, 0.0, comm_ref[1])

        def cstep(t, g):
            g = g * dA[None]
            C_t = C_ref[:, pl.ds(t, 1), :][:, 0, :]
            y_t = y_ref[:, pl.ds(t, 1), :][:, 0, :]
            y_ref[:, pl.ds(t, 1), :] = (
                y_t + jnp.sum(g * C_t[:, :, None], axis=1)
            )[:, None, :]
            return g

        lax.fori_loop(0, TC, cstep, h_in)

    return pl.pallas_call(
        body,
        out_shape=jax.ShapeDtypeStruct((Bb, S, D), jnp.float32),
        in_specs=[
            pl.BlockSpec(memory_space=pltpu.VMEM),
            pl.BlockSpec(memory_space=pltpu.VMEM),
            pl.BlockSpec(memory_space=pltpu.VMEM),
            pl.BlockSpec(memory_space=pltpu.VMEM),
        ],
        out_specs=pl.BlockSpec(memory_space=pltpu.VMEM),
        scratch_shapes=[
            pltpu.VMEM((2, Bb, N, D), jnp.float32),
            pltpu.SemaphoreType.DMA,
            pltpu.SemaphoreType.DMA,
        ],
    )(x, A, B, C)
